# Initial kernel scaffold; baseline (speedup 1.0000x reference)
import jax
import jax.numpy as jnp
from jax.experimental import pallas as pl


def kernel(input_tensor, expert_sel, keys, values, bias):
    raise NotImplementedError("write your pallas kernel here")


# speedup vs baseline: 2.2796x; 2.2796x over previous
import functools

import jax
import jax.numpy as jnp
from jax.experimental import pallas as pl


def _moe_body(x_ref, es_ref, k_ref, v_ref, o_ref):
    e = pl.program_id(1)
    xb = x_ref[...]
    g = jax.nn.sigmoid(
        jax.lax.dot_general(
            xb, es_ref[0], (((1,), (1,)), ((), ())),
            preferred_element_type=jnp.float32,
        )
    )
    h = jnp.maximum(
        jnp.dot(xb, k_ref[0], preferred_element_type=jnp.float32), 0.0
    )
    contrib = jnp.dot(h * g, v_ref[0], preferred_element_type=jnp.float32)

    @pl.when(e == 0)
    def _init():
        o_ref[...] = contrib

    @pl.when(e != 0)
    def _acc():
        o_ref[...] += contrib


@functools.partial(jax.jit, static_argnames=("tm",))
def _moe(x, es3, keys, values, tm):
    t, d = x.shape
    e, _, h = keys.shape
    out = pl.pallas_call(
        _moe_body,
        grid=(t // tm, e),
        in_specs=[
            pl.BlockSpec((tm, d), lambda i, j: (i, 0)),
            pl.BlockSpec((1, 1, d), lambda i, j: (j, 0, 0)),
            pl.BlockSpec((1, d, h), lambda i, j: (j, 0, 0)),
            pl.BlockSpec((1, h, d), lambda i, j: (j, 0, 0)),
        ],
        out_specs=pl.BlockSpec((tm, d), lambda i, j: (i, 0)),
        out_shape=jax.ShapeDtypeStruct((t, d), jnp.float32),
    )(x, es3, keys, values)
    return out


def kernel(input_tensor, expert_sel, keys, values, bias):
    b, s, d = input_tensor.shape
    n_exp = keys.shape[0]
    x = input_tensor.reshape(b * s, d)
    es3 = expert_sel.reshape(n_exp, 1, d)
    out = _moe(x, es3, keys, values, tm=512)
    return out.reshape(b, s, d)


# speedup vs baseline: 2.7790x; 1.2191x over previous
import functools

import jax
import jax.numpy as jnp
from jax.experimental import pallas as pl


def _moe_body(x_ref, es_ref, k_ref, v_ref, o_ref):
    e = pl.program_id(1)
    xb = x_ref[...]
    g = jax.nn.sigmoid(
        jax.lax.dot_general(
            xb, es_ref[0], (((1,), (1,)), ((), ())),
            preferred_element_type=jnp.float32,
        )
    )
    h = jnp.maximum(
        jnp.dot(xb, k_ref[0], preferred_element_type=jnp.float32), 0.0
    )
    contrib = jnp.dot(h * g, v_ref[0], preferred_element_type=jnp.float32)

    @pl.when(e == 0)
    def _init():
        o_ref[...] = contrib

    @pl.when(e != 0)
    def _acc():
        o_ref[...] += contrib


@functools.partial(jax.jit, static_argnames=("tm",))
def _moe(x, es3, keys, values, tm):
    t, d = x.shape
    e, _, h = keys.shape
    out = pl.pallas_call(
        _moe_body,
        grid=(t // tm, e),
        in_specs=[
            pl.BlockSpec((tm, d), lambda i, j: (i, 0)),
            pl.BlockSpec((1, 1, d), lambda i, j: (j, 0, 0)),
            pl.BlockSpec((1, d, h), lambda i, j: (j, 0, 0)),
            pl.BlockSpec((1, h, d), lambda i, j: (j, 0, 0)),
        ],
        out_specs=pl.BlockSpec((tm, d), lambda i, j: (i, 0)),
        out_shape=jax.ShapeDtypeStruct((t, d), jnp.float32),
    )(x, es3, keys, values)
    return out


def kernel(input_tensor, expert_sel, keys, values, bias):
    b, s, d = input_tensor.shape
    n_exp = keys.shape[0]
    x = input_tensor.reshape(b * s, d)
    es3 = expert_sel.reshape(n_exp, 1, d)
    out = _moe(x, es3, keys, values, tm=1024)
    return out.reshape(b, s, d)
